# Initial kernel scaffold; baseline (speedup 1.0000x reference)
import jax
import jax.numpy as jnp
from jax.experimental import pallas as pl


def kernel(x, weight, band_mask):
    raise NotImplementedError("write your pallas kernel here")


# speedup vs baseline: 2.9833x; 2.9833x over previous
import functools

import jax
import jax.numpy as jnp
from jax import lax
from jax.experimental import pallas as pl
from jax.experimental.pallas import tpu as pltpu
from jax.experimental.pallas import tpu_sc as plsc

NUM_CORES = 2
NUM_SUBCORES = 16
NUM_WORKERS = NUM_CORES * NUM_SUBCORES
CHUNK = 256
DMA_IDX = 128
LANES = 16


@functools.lru_cache(maxsize=None)
def _build_sc_gather(B, D, b_per_w):
    n_chunks = b_per_w // CHUNK
    mesh = plsc.VectorSubcoreMesh(core_axis_name="c", subcore_axis_name="s")

    @functools.partial(
        pl.kernel,
        mesh=mesh,
        out_type=jax.ShapeDtypeStruct((B, D), jnp.float32),
        scratch_types=[
            pltpu.VMEM((b_per_w,), jnp.int32),
            pltpu.VMEM((D,), jnp.float32),
            pltpu.VMEM((CHUNK, D), jnp.float32),
            pltpu.SemaphoreType.DMA,
        ],
    )
    def k(idx_hbm, table_hbm, mask_hbm, out_hbm, idx_v, mask_v, rows_v, sem):
        wid = lax.axis_index("s") * NUM_CORES + lax.axis_index("c")
        base = pl.multiple_of(wid * b_per_w, 8)
        pltpu.sync_copy(idx_hbm.at[pl.ds(base, b_per_w)], idx_v)
        pltpu.sync_copy(mask_hbm, mask_v)
        mvs = [mask_v[pl.ds(h * LANES, LANES)] for h in range(D // LANES)]

        def chunk_body(c, carry):
            off = pl.multiple_of(c * CHUNK, 8)
            copies = []
            for j in range(CHUNK // DMA_IDX):
                copies.append(pltpu.async_copy(
                    table_hbm.at[idx_v.at[pl.ds(off + j * DMA_IDX, DMA_IDX)]],
                    rows_v.at[pl.ds(j * DMA_IDX, DMA_IDX)],
                    sem,
                ))
            for cp in copies:
                cp.wait()

            def row_body(r, rcarry):
                for h in range(D // LANES):
                    sl = pl.ds(h * LANES, LANES)
                    rows_v[r, sl] = rows_v[r, sl] * mvs[h]
                return rcarry

            lax.fori_loop(0, CHUNK, row_body, 0, unroll=False)
            pltpu.sync_copy(rows_v, out_hbm.at[pl.ds(base + off, CHUNK)])
            return carry

        lax.fori_loop(0, n_chunks, chunk_body, 0, unroll=False)

    return k


def kernel(x, weight, band_mask):
    Bdim0, Bdim1 = x.shape
    B = Bdim0 * Bdim1
    D = weight.shape[1]
    b_per_w = B // NUM_WORKERS
    idx = x.reshape(B).astype(jnp.int32)
    out = _build_sc_gather(B, D, b_per_w)(idx, weight, band_mask)
    return out.reshape(Bdim0, Bdim1, D)
